# baseline (speedup 1.0000x reference)
import functools
import math

import jax
import jax.numpy as jnp
from jax import lax
from jax.experimental import pallas as pl
from jax.experimental.pallas import tpu as pltpu
from jax.experimental.pallas import tpu_sc as plsc

_BATCH = 1024
_VOCAB = 65536
_CODE_LEN = 16
_LANES = 16
_NUM_CORES = 2
_NUM_SUBCORES = 16
_NUM_WORKERS = _NUM_CORES * _NUM_SUBCORES

_SC_ROWS = 512
_TC_ROWS = _BATCH - _SC_ROWS
_ROWS_PER_W = _SC_ROWS // _NUM_WORKERS

_BLOCKS = sorted(
    {((1 << k) - 1) >> 7 for k in range(_CODE_LEN)}
    | {((1 << k) - 1 + k) >> 7 for k in range(_CODE_LEN)}
)
_SLOT = {blk: i for i, blk in enumerate(_BLOCKS)}
_NUM_BLOCKS = len(_BLOCKS)
_STAGE_COLS = _NUM_BLOCKS * 128

_RUNS = []
for _b in _BLOCKS:
    if _RUNS and _RUNS[-1][0] + _RUNS[-1][1] == _b:
        _RUNS[-1][1] += 1
    else:
        _RUNS.append([_b, 1])
_RUNS = [tuple(r) for r in _RUNS]
_NUM_RUNS = len(_RUNS)

_LEVEL_RUN = []
for _k in range(_CODE_LEN):
    _lo = ((1 << _k) - 1) >> 7
    _hi = ((1 << _k) - 1 + _k) >> 7
    (_ri,) = [i for i, (s, n) in enumerate(_RUNS) if s <= _lo and _hi < s + n]
    _LEVEL_RUN.append(_ri)

_LN2 = math.log(2.0)


def _neg_log(p):
    p = jnp.maximum(p, jnp.float32(2.0**-126))
    bits = plsc.bitcast(p, jnp.int32)
    e = ((bits >> 23) & 255) - 127
    m = plsc.bitcast((bits & 0x007FFFFF) | 0x3F800000, jnp.float32)
    t = (m - 1.0) / (m + 1.0)
    t2 = t * t
    c1 = jnp.float32(2.0 / _LN2)
    c3 = jnp.float32(2.0 / (3.0 * _LN2))
    c5 = jnp.float32(2.0 / (5.0 * _LN2))
    c7 = jnp.float32(2.0 / (7.0 * _LN2))
    log2m = t * (c1 + t2 * (c3 + t2 * (c5 + t2 * c7)))
    return -jnp.float32(_LN2) * (e.astype(jnp.float32) + log2m)


def _start_run_copies(scores_ref, vals_ref, row0, nrows, sems):
    copies = []
    for ri, (blk0, nblk) in enumerate(_RUNS):
        cp = pltpu.make_async_copy(
            scores_ref.at[pl.ds(row0, nrows), pl.ds(blk0 * 128, nblk * 128)],
            vals_ref.at[:, pl.ds(_SLOT[blk0] * 128, nblk * 128)],
            sems.at[ri],
        )
        cp.start()
        copies.append(cp)
    return copies


def _sc_body(scores_hbm, cls_hbm, part_hbm, cls_v, vals_v, out_v, sems):
    cid = lax.axis_index("c")
    sid = lax.axis_index("s")
    wid = sid * _NUM_CORES + cid
    base = wid * _ROWS_PER_W

    pltpu.sync_copy(cls_hbm.at[pl.ds(base, _ROWS_PER_W)], cls_v)
    rows = lax.iota(jnp.int32, _LANES)
    copies = _start_run_copies(scores_hbm, vals_v, base, _ROWS_PER_W, sems)

    one = jnp.float32(1.0)
    c = cls_v[...]
    num_acc = jnp.ones((_LANES,), jnp.float32)
    den_acc = jnp.ones((_LANES,), jnp.float32)
    prefix = jnp.zeros((_LANES,), jnp.int32)
    waited = set()
    for k in range(_CODE_LEN):
        ri = _LEVEL_RUN[k]
        if ri not in waited:
            copies[ri].wait()
            waited.add(ri)
        bit = (c >> (_CODE_LEN - 1 - k)) & 1
        col = ((1 << k) - 1) + prefix
        lo_blk = ((1 << k) - 1) >> 7
        slot = _SLOT[lo_blk] + ((col >> 7) - lo_blk)
        s = plsc.load_gather(vals_v, [rows, slot * 128 + (col & 127)])
        z = jnp.where(bit == 1, -s, s)
        e = jnp.exp(-jnp.abs(z))
        num_acc = num_acc * jnp.where(z >= 0, one, e)
        den_acc = den_acc * (one + e)
        prefix = prefix + bit

    out_v[...] = _neg_log(num_acc / den_acc)
    pltpu.sync_copy(out_v, part_hbm.at[wid])


@functools.cache
def _sc_loss_parts():
    return pl.kernel(
        _sc_body,
        mesh=plsc.VectorSubcoreMesh(core_axis_name="c", subcore_axis_name="s"),
        out_type=jax.ShapeDtypeStruct((_NUM_WORKERS, _LANES), jnp.float32),
        compiler_params=pltpu.CompilerParams(needs_layout_passes=False),
        scratch_types=[
            pltpu.VMEM((_ROWS_PER_W,), jnp.int32),
            pltpu.VMEM((_ROWS_PER_W, _STAGE_COLS), jnp.float32),
            pltpu.VMEM((_LANES,), jnp.float32),
            pltpu.SemaphoreType.DMA((_NUM_RUNS,)),
        ],
    )


def _tc_body(scores_any, cls_any, out_ref, vals_v, cls_v, sems, csem):
    cp_cls = pltpu.make_async_copy(
        cls_any.at[pl.ds(_SC_ROWS, _TC_ROWS)], cls_v, csem
    )
    cp_cls.start()
    copies = _start_run_copies(scores_any, vals_v, _SC_ROWS, _TC_ROWS, sems)
    cp_cls.wait()

    c = cls_v[...][:, None]
    lane = lax.broadcasted_iota(jnp.int32, (_TC_ROWS, 128), 1)
    zero = jnp.zeros((_TC_ROWS, 128), jnp.float32)
    loss = jnp.zeros((_TC_ROWS, 1), jnp.float32)
    prefix = jnp.zeros((_TC_ROWS, 1), jnp.int32)
    waited = set()
    for k in range(_CODE_LEN):
        ri = _LEVEL_RUN[k]
        if ri not in waited:
            copies[ri].wait()
            waited.add(ri)
        bit = (c >> (_CODE_LEN - 1 - k)) & 1
        col = ((1 << k) - 1) + prefix
        lo_blk = ((1 << k) - 1) >> 7
        lo_slot = _SLOT[lo_blk]
        hi = (col >> 7) - lo_blk
        colmod = col & 127
        m_lo = (lane == colmod) & (hi == 0)
        sel = jnp.where(m_lo, vals_v[:, pl.ds(lo_slot * 128, 128)], zero)
        if ((1 << k) - 1 + k) >> 7 > lo_blk:
            m_hi = (lane == colmod) & (hi == 1)
            sel = sel + jnp.where(
                m_hi, vals_v[:, pl.ds((lo_slot + 1) * 128, 128)], zero
            )
        s = jnp.sum(sel, axis=1, keepdims=True)
        z = jnp.where(bit == 1, -s, s)
        loss = loss + jnp.maximum(-z, 0.0) + jnp.log(1.0 + jnp.exp(-jnp.abs(z)))
        prefix = prefix + bit

    out_ref[0, 0] = jnp.sum(loss)


@functools.cache
def _tc_loss_part():
    return pl.pallas_call(
        _tc_body,
        out_shape=jax.ShapeDtypeStruct((1, 1), jnp.float32),
        in_specs=[
            pl.BlockSpec(memory_space=pl.ANY),
            pl.BlockSpec(memory_space=pl.ANY),
        ],
        out_specs=pl.BlockSpec(memory_space=pltpu.SMEM),
        scratch_shapes=[
            pltpu.VMEM((_TC_ROWS, _STAGE_COLS), jnp.float32),
            pltpu.VMEM((_TC_ROWS,), jnp.int32),
            pltpu.SemaphoreType.DMA((_NUM_RUNS,)),
            pltpu.SemaphoreType.DMA,
        ],
    )


def kernel(scores, class_indices):
    tc_part = _tc_loss_part()(scores, class_indices)
    sc_parts = _sc_loss_parts()(scores, class_indices)
    total = jnp.sum(sc_parts) + tc_part[0, 0]
    return total * jnp.float32(1.0 / _BATCH)


# speedup vs baseline: 1.0203x; 1.0203x over previous
import functools
import math

import jax
import jax.numpy as jnp
from jax import lax
from jax.experimental import pallas as pl
from jax.experimental.pallas import tpu as pltpu
from jax.experimental.pallas import tpu_sc as plsc

_BATCH = 1024
_VOCAB = 65536
_CODE_LEN = 16
_LANES = 16
_NUM_CORES = 2
_NUM_SUBCORES = 16
_NUM_WORKERS = _NUM_CORES * _NUM_SUBCORES

_SC_ROWS = 512
_TC_ROWS = _BATCH - _SC_ROWS
_ROWS_PER_W = _SC_ROWS // _NUM_WORKERS

_BLOCKS = sorted(
    {((1 << k) - 1) >> 7 for k in range(_CODE_LEN)}
    | {((1 << k) - 1 + k) >> 7 for k in range(_CODE_LEN)}
)
_SLOT = {blk: i for i, blk in enumerate(_BLOCKS)}
_NUM_BLOCKS = len(_BLOCKS)
_STAGE_COLS = _NUM_BLOCKS * 128

_RUNS = []
for _b in _BLOCKS:
    if _RUNS and _RUNS[-1][0] + _RUNS[-1][1] == _b:
        _RUNS[-1][1] += 1
    else:
        _RUNS.append([_b, 1])
_RUNS = [tuple(r) for r in _RUNS]
_NUM_RUNS = len(_RUNS)

_LEVEL_RUN = []
for _k in range(_CODE_LEN):
    _lo = ((1 << _k) - 1) >> 7
    _hi = ((1 << _k) - 1 + _k) >> 7
    (_ri,) = [i for i, (s, n) in enumerate(_RUNS) if s <= _lo and _hi < s + n]
    _LEVEL_RUN.append(_ri)

_LN2 = math.log(2.0)


def _neg_log(p):
    p = jnp.maximum(p, jnp.float32(2.0**-126))
    bits = plsc.bitcast(p, jnp.int32)
    e = ((bits >> 23) & 255) - 127
    m = plsc.bitcast((bits & 0x007FFFFF) | 0x3F800000, jnp.float32)
    t = (m - 1.0) / (m + 1.0)
    t2 = t * t
    c1 = jnp.float32(2.0 / _LN2)
    c3 = jnp.float32(2.0 / (3.0 * _LN2))
    c5 = jnp.float32(2.0 / (5.0 * _LN2))
    c7 = jnp.float32(2.0 / (7.0 * _LN2))
    log2m = t * (c1 + t2 * (c3 + t2 * (c5 + t2 * c7)))
    return -jnp.float32(_LN2) * (e.astype(jnp.float32) + log2m)


def _start_run_copies(scores_ref, vals_ref, row0, nrows, sems):
    copies = []
    for ri, (blk0, nblk) in enumerate(_RUNS):
        cp = pltpu.make_async_copy(
            scores_ref.at[pl.ds(row0, nrows), pl.ds(blk0 * 128, nblk * 128)],
            vals_ref.at[:, pl.ds(_SLOT[blk0] * 128, nblk * 128)],
            sems.at[ri],
        )
        cp.start()
        copies.append(cp)
    return copies


def _sc_body(scores_hbm, cls_hbm, part_hbm, cls_v, vals_v, out_v, sems):
    cid = lax.axis_index("c")
    sid = lax.axis_index("s")
    wid = sid * _NUM_CORES + cid
    base = wid * _ROWS_PER_W

    rows = lax.iota(jnp.int32, _LANES)
    copies = _start_run_copies(scores_hbm, vals_v, base, _ROWS_PER_W, sems)
    pltpu.sync_copy(cls_hbm.at[pl.ds(base, _ROWS_PER_W)], cls_v)

    one = jnp.float32(1.0)
    c = cls_v[...]
    num_acc = jnp.ones((_LANES,), jnp.float32)
    den_acc = jnp.ones((_LANES,), jnp.float32)
    prefix = jnp.zeros((_LANES,), jnp.int32)
    waited = set()
    for k in range(_CODE_LEN):
        ri = _LEVEL_RUN[k]
        if ri not in waited:
            copies[ri].wait()
            waited.add(ri)
        bit = (c >> (_CODE_LEN - 1 - k)) & 1
        col = ((1 << k) - 1) + prefix
        lo_blk = ((1 << k) - 1) >> 7
        slot = _SLOT[lo_blk] + ((col >> 7) - lo_blk)
        s = plsc.load_gather(vals_v, [rows, slot * 128 + (col & 127)])
        z = jnp.where(bit == 1, -s, s)
        e = jnp.exp(-jnp.abs(z))
        num_acc = num_acc * jnp.where(z >= 0, one, e)
        den_acc = den_acc * (one + e)
        prefix = prefix + bit

    out_v[...] = _neg_log(num_acc / den_acc)
    pltpu.sync_copy(out_v, part_hbm.at[wid])


@functools.cache
def _sc_loss_parts():
    return pl.kernel(
        _sc_body,
        mesh=plsc.VectorSubcoreMesh(core_axis_name="c", subcore_axis_name="s"),
        out_type=jax.ShapeDtypeStruct((_NUM_WORKERS, _LANES), jnp.float32),
        compiler_params=pltpu.CompilerParams(needs_layout_passes=False),
        scratch_types=[
            pltpu.VMEM((_ROWS_PER_W,), jnp.int32),
            pltpu.VMEM((_ROWS_PER_W, _STAGE_COLS), jnp.float32),
            pltpu.VMEM((_LANES,), jnp.float32),
            pltpu.SemaphoreType.DMA((_NUM_RUNS,)),
        ],
    )


def _tc_body(scores_any, cls_any, out_ref, vals_v, cls_v, sems, csem):
    cp_cls = pltpu.make_async_copy(
        cls_any.at[pl.ds(_SC_ROWS, _TC_ROWS)], cls_v, csem
    )
    cp_cls.start()
    copies = _start_run_copies(scores_any, vals_v, _SC_ROWS, _TC_ROWS, sems)
    cp_cls.wait()

    c = cls_v[...][:, None]
    lane = lax.broadcasted_iota(jnp.int32, (_TC_ROWS, 128), 1)
    zero = jnp.zeros((_TC_ROWS, 128), jnp.float32)
    loss = jnp.zeros((_TC_ROWS, 1), jnp.float32)
    prefix = jnp.zeros((_TC_ROWS, 1), jnp.int32)
    waited = set()
    for k in range(_CODE_LEN):
        ri = _LEVEL_RUN[k]
        if ri not in waited:
            copies[ri].wait()
            waited.add(ri)
        bit = (c >> (_CODE_LEN - 1 - k)) & 1
        col = ((1 << k) - 1) + prefix
        lo_blk = ((1 << k) - 1) >> 7
        lo_slot = _SLOT[lo_blk]
        hi = (col >> 7) - lo_blk
        colmod = col & 127
        m_lo = (lane == colmod) & (hi == 0)
        sel = jnp.where(m_lo, vals_v[:, pl.ds(lo_slot * 128, 128)], zero)
        if ((1 << k) - 1 + k) >> 7 > lo_blk:
            m_hi = (lane == colmod) & (hi == 1)
            sel = sel + jnp.where(
                m_hi, vals_v[:, pl.ds((lo_slot + 1) * 128, 128)], zero
            )
        s = jnp.sum(sel, axis=1, keepdims=True)
        z = jnp.where(bit == 1, -s, s)
        loss = loss + jnp.maximum(-z, 0.0) + jnp.log(1.0 + jnp.exp(-jnp.abs(z)))
        prefix = prefix + bit

    out_ref[0, 0] = jnp.sum(loss)


@functools.cache
def _tc_loss_part():
    return pl.pallas_call(
        _tc_body,
        out_shape=jax.ShapeDtypeStruct((1, 1), jnp.float32),
        in_specs=[
            pl.BlockSpec(memory_space=pl.ANY),
            pl.BlockSpec(memory_space=pl.ANY),
        ],
        out_specs=pl.BlockSpec(memory_space=pltpu.SMEM),
        scratch_shapes=[
            pltpu.VMEM((_TC_ROWS, _STAGE_COLS), jnp.float32),
            pltpu.VMEM((_TC_ROWS,), jnp.int32),
            pltpu.SemaphoreType.DMA((_NUM_RUNS,)),
            pltpu.SemaphoreType.DMA,
        ],
    )


def kernel(scores, class_indices):
    tc_part = _tc_loss_part()(scores, class_indices)
    sc_parts = _sc_loss_parts()(scores, class_indices)
    total = jnp.sum(sc_parts) + tc_part[0, 0]
    return total * jnp.float32(1.0 / _BATCH)


# speedup vs baseline: 1.0311x; 1.0106x over previous
import functools
import math

import jax
import jax.numpy as jnp
from jax import lax
from jax.experimental import pallas as pl
from jax.experimental.pallas import tpu as pltpu
from jax.experimental.pallas import tpu_sc as plsc

_BATCH = 1024
_VOCAB = 65536
_CODE_LEN = 16
_LANES = 16
_NUM_CORES = 2
_NUM_SUBCORES = 16
_NUM_WORKERS = _NUM_CORES * _NUM_SUBCORES

_SC_ROWS = 512
_TC_ROWS = _BATCH - _SC_ROWS
_ROWS_PER_W = _SC_ROWS // _NUM_WORKERS

_BLOCKS = sorted(
    {((1 << k) - 1) >> 7 for k in range(_CODE_LEN)}
    | {((1 << k) - 1 + k) >> 7 for k in range(_CODE_LEN)}
)
_SLOT = {blk: i for i, blk in enumerate(_BLOCKS)}
_NUM_BLOCKS = len(_BLOCKS)

_LN2 = math.log(2.0)


def _neg_log(p):
    p = jnp.maximum(p, jnp.float32(2.0**-126))
    bits = plsc.bitcast(p, jnp.int32)
    e = ((bits >> 23) & 255) - 127
    m = plsc.bitcast((bits & 0x007FFFFF) | 0x3F800000, jnp.float32)
    t = (m - 1.0) / (m + 1.0)
    t2 = t * t
    c1 = jnp.float32(2.0 / _LN2)
    c3 = jnp.float32(2.0 / (3.0 * _LN2))
    c5 = jnp.float32(2.0 / (5.0 * _LN2))
    c7 = jnp.float32(2.0 / (7.0 * _LN2))
    log2m = t * (c1 + t2 * (c3 + t2 * (c5 + t2 * c7)))
    return -jnp.float32(_LN2) * (e.astype(jnp.float32) + log2m)


def _sc_body(scores_hbm, cls_hbm, part_hbm, cls_v, vals_v, out_v, sem):
    cid = lax.axis_index("c")
    sid = lax.axis_index("s")
    wid = sid * _NUM_CORES + cid
    base = wid * _ROWS_PER_W

    rows = lax.iota(jnp.int32, _LANES)

    copies = []
    for slot, blk in enumerate(_BLOCKS):
        cp = pltpu.make_async_copy(
            scores_hbm.at[pl.ds(base, _ROWS_PER_W), pl.ds(blk * 128, 128)],
            vals_v.at[slot],
            sem,
        )
        cp.start()
        copies.append(cp)
    pltpu.sync_copy(cls_hbm.at[pl.ds(base, _ROWS_PER_W)], cls_v)
    for cp in copies:
        cp.wait()

    one = jnp.float32(1.0)
    c = cls_v[...]
    num_acc = jnp.ones((_LANES,), jnp.float32)
    den_acc = jnp.ones((_LANES,), jnp.float32)
    prefix = jnp.zeros((_LANES,), jnp.int32)
    for k in range(_CODE_LEN):
        bit = (c >> (_CODE_LEN - 1 - k)) & 1
        col = ((1 << k) - 1) + prefix
        lo_blk = ((1 << k) - 1) >> 7
        slot = _SLOT[lo_blk] + ((col >> 7) - lo_blk)
        s = plsc.load_gather(vals_v, [slot, rows, col & 127])
        z = jnp.where(bit == 1, -s, s)
        e = jnp.exp(-jnp.abs(z))
        num_acc = num_acc * jnp.where(z >= 0, one, e)
        den_acc = den_acc * (one + e)
        prefix = prefix + bit

    out_v[...] = _neg_log(num_acc / den_acc)
    pltpu.sync_copy(out_v, part_hbm.at[wid])


@functools.cache
def _sc_loss_parts():
    return pl.kernel(
        _sc_body,
        mesh=plsc.VectorSubcoreMesh(core_axis_name="c", subcore_axis_name="s"),
        out_type=jax.ShapeDtypeStruct((_NUM_WORKERS, _LANES), jnp.float32),
        compiler_params=pltpu.CompilerParams(needs_layout_passes=False),
        scratch_types=[
            pltpu.VMEM((_ROWS_PER_W,), jnp.int32),
            pltpu.VMEM((_NUM_BLOCKS, _ROWS_PER_W, 128), jnp.float32),
            pltpu.VMEM((_LANES,), jnp.float32),
            pltpu.SemaphoreType.DMA,
        ],
    )


def _tc_body(scores_any, cls_any, out_ref, vals_v, cls_v, sem, csem):
    cp_cls = pltpu.make_async_copy(
        cls_any.at[pl.ds(_SC_ROWS, _TC_ROWS)], cls_v, csem
    )
    cp_cls.start()
    copies = []
    for slot, blk in enumerate(_BLOCKS):
        cp = pltpu.make_async_copy(
            scores_any.at[pl.ds(_SC_ROWS, _TC_ROWS), pl.ds(blk * 128, 128)],
            vals_v.at[slot],
            sem,
        )
        cp.start()
        copies.append(cp)
    cp_cls.wait()
    for cp in copies:
        cp.wait()

    c = cls_v[...][:, None]
    lane = lax.broadcasted_iota(jnp.int32, (_TC_ROWS, 128), 1)
    zero = jnp.zeros((_TC_ROWS, 128), jnp.float32)
    loss = jnp.zeros((_TC_ROWS, 1), jnp.float32)
    prefix = jnp.zeros((_TC_ROWS, 1), jnp.int32)
    for k in range(_CODE_LEN):
        bit = (c >> (_CODE_LEN - 1 - k)) & 1
        col = ((1 << k) - 1) + prefix
        lo_blk = ((1 << k) - 1) >> 7
        lo_slot = _SLOT[lo_blk]
        hi = (col >> 7) - lo_blk
        colmod = col & 127
        m_lo = (lane == colmod) & (hi == 0)
        sel = jnp.where(m_lo, vals_v[lo_slot], zero)
        if ((1 << k) - 1 + k) >> 7 > lo_blk:
            m_hi = (lane == colmod) & (hi == 1)
            sel = sel + jnp.where(m_hi, vals_v[lo_slot + 1], zero)
        s = jnp.sum(sel, axis=1, keepdims=True)
        z = jnp.where(bit == 1, -s, s)
        loss = loss + jnp.maximum(-z, 0.0) + jnp.log(1.0 + jnp.exp(-jnp.abs(z)))
        prefix = prefix + bit

    out_ref[0, 0] = jnp.sum(loss)


@functools.cache
def _tc_loss_part():
    return pl.pallas_call(
        _tc_body,
        out_shape=jax.ShapeDtypeStruct((1, 1), jnp.float32),
        in_specs=[
            pl.BlockSpec(memory_space=pl.ANY),
            pl.BlockSpec(memory_space=pl.ANY),
        ],
        out_specs=pl.BlockSpec(memory_space=pltpu.SMEM),
        scratch_shapes=[
            pltpu.VMEM((_NUM_BLOCKS, _TC_ROWS, 128), jnp.float32),
            pltpu.VMEM((_TC_ROWS,), jnp.int32),
            pltpu.SemaphoreType.DMA,
            pltpu.SemaphoreType.DMA,
        ],
    )


def kernel(scores, class_indices):
    tc_part = _tc_loss_part()(scores, class_indices)
    sc_parts = _sc_loss_parts()(scores, class_indices)
    total = jnp.sum(sc_parts) + tc_part[0, 0]
    return total * jnp.float32(1.0 / _BATCH)
